# baseline (speedup 1.0000x reference)
import functools

import jax
import jax.numpy as jnp
from jax import lax
from jax.experimental import pallas as pl
from jax.experimental.pallas import tpu as pltpu
from jax.experimental.pallas import tpu_sc as plsc

N, K, C = 10000, 32, 128
NC, NS = 2, 16
NW = NC * NS
CP2 = C // 2
CW = CP2 // NW
K2 = K // 2
CHP = 400
NCHK = N // CHP
PB = CHP // 16
IDX_PER_W = (N * K) // NW
EPS = 1e-5


def _wid():
    return lax.axis_index("s") * NC + lax.axis_index("c")


@functools.lru_cache(maxsize=None)
def _build_sc_kernels():
    mesh = plsc.VectorSubcoreMesh(core_axis_name="c", subcore_axis_name="s")
    cparams = pltpu.CompilerParams(needs_layout_passes=False)

    @functools.partial(
        pl.kernel,
        out_type=jax.ShapeDtypeStruct((NW, N), jnp.float32),
        mesh=mesh,
        compiler_params=cparams,
        scratch_types=[
            pltpu.VMEM((IDX_PER_W,), jnp.int32),
            pltpu.VMEM((N,), jnp.float32),
        ],
    )
    def bincount(idx_hbm, pc_hbm, idx_v, cnt_v):
        w = _wid()
        pltpu.sync_copy(idx_hbm.at[w], idx_v)
        zeros = jnp.zeros((16,), jnp.float32)

        def zbody(i, c):
            cnt_v[pl.ds(i * 16, 16)] = zeros
            return c

        lax.fori_loop(0, N // 16, zbody, 0)
        ones = jnp.ones((16,), jnp.float32)

        def abody(i, c):
            v = idx_v[pl.ds(i * 16, 16)]
            plsc.addupdate_scatter(cnt_v, [v], ones)
            return c

        lax.fori_loop(0, IDX_PER_W // 16, abody, 0)
        pltpu.sync_copy(cnt_v, pc_hbm.at[w])

    @functools.partial(
        pl.kernel,
        out_type=jax.ShapeDtypeStruct((CP2, N), jnp.float32),
        mesh=mesh,
        compiler_params=cparams,
        scratch_types=[
            pltpu.VMEM((CW, N), jnp.float32),
            pltpu.VMEM((K2, CHP), jnp.int32),
            pltpu.VMEM((K2, CHP), jnp.int32),
            pltpu.VMEM((CW, N), jnp.float32),
            pltpu.SemaphoreType.DMA,
            pltpu.SemaphoreType.DMA,
        ],
    )
    def gather_max(tblT_hbm, idxc_hbm, outT_hbm, tbl_v, idx0, idx1, out_v,
                   sem0, sem1):
        w = _wid()
        pltpu.sync_copy(tblT_hbm.at[pl.ds(w * CW, CW)], tbl_v)
        pltpu.async_copy(idxc_hbm.at[0], idx0, sem0)
        cvecs = [jnp.full((16,), c, jnp.int32) for c in range(CW)]

        def compute(idx_v, ch):
            n0 = ch * CHP

            mask16 = jnp.full((16,), 0xFFFF, jnp.int32)

            def pbbody(pb, c):
                col = pb * 16
                accs = None
                for k2 in range(K2):
                    vk = idx_v[k2, pl.ds(col, 16)]
                    vlo = jnp.bitwise_and(vk, mask16)
                    vhi = lax.shift_right_logical(vk, 16)
                    for vi in (vlo, vhi):
                        if accs is None:
                            accs = [
                                plsc.bitcast(
                                    plsc.load_gather(tbl_v, [cvecs[c4], vi]),
                                    jnp.bfloat16)
                                for c4 in range(CW)
                            ]
                        else:
                            for c4 in range(CW):
                                g = plsc.bitcast(
                                    plsc.load_gather(tbl_v, [cvecs[c4], vi]),
                                    jnp.bfloat16)
                                accs[c4] = jnp.maximum(accs[c4], g)
                for c4 in range(CW):
                    out_v[c4, pl.ds(n0 + col, 16)] = plsc.bitcast(
                        accs[c4], jnp.float32)
                return c

            lax.fori_loop(0, PB, pbbody, 0)

        def jbody(j, c):
            ch0 = 2 * j
            ch1 = 2 * j + 1
            pltpu.make_async_copy(idxc_hbm.at[ch0], idx0, sem0).wait()
            pltpu.async_copy(idxc_hbm.at[ch1], idx1, sem1)
            compute(idx0, ch0)
            pltpu.async_copy(idxc_hbm.at[ch0 + 2], idx0, sem0)
            pltpu.make_async_copy(idxc_hbm.at[ch1], idx1, sem1).wait()
            compute(idx1, ch1)
            return c

        lax.fori_loop(0, NCHK // 2, jbody, 0)
        pltpu.make_async_copy(idxc_hbm.at[NCHK - 1], idx0, sem0).wait()
        compute(idx0, NCHK - 1)
        pltpu.sync_copy(out_v, outT_hbm.at[pl.ds(w * CW, CW)])

    return bincount, gather_max


def _bn_relu(y, wrow, g, b):
    m = jnp.dot(wrow, y, preferred_element_type=jnp.float32,
                precision=lax.Precision.HIGHEST)
    q = jnp.dot(wrow, y * y, preferred_element_type=jnp.float32,
                precision=lax.Precision.HIGHEST)
    v = q - m * m
    scale = lax.rsqrt(v + EPS) * g
    shift = b - m * scale
    return jnp.maximum(y * scale + shift, 0.0)


def _mlp_body(pc_ref, feat_ref, fc1_ref, l1w_ref, l1b_ref, l2w_ref, l2b_ref,
              n1g_ref, n1b_ref, m1g_ref, m1b_ref, m2g_ref, m2b_ref, out_ref):
    f = feat_ref[...]
    urow = jnp.full((1, N), 1.0 / N, jnp.float32)
    y0 = jnp.dot(f, fc1_ref[...], preferred_element_type=jnp.float32)
    x1 = _bn_relu(y0, urow, n1g_ref[...], n1b_ref[...])

    wrow = jnp.sum(pc_ref[...], axis=0, keepdims=True) * (1.0 / (N * K))

    y1 = jnp.dot(x1, l1w_ref[...], preferred_element_type=jnp.float32) + l1b_ref[...]
    h1 = _bn_relu(y1, wrow, m1g_ref[...], m1b_ref[...])

    y2 = jnp.dot(h1, l2w_ref[...], preferred_element_type=jnp.float32) + l2b_ref[...]
    h2 = _bn_relu(y2, wrow, m2g_ref[...], m2b_ref[...])
    lo = lax.bitcast_convert_type(
        h2[:, :CP2].astype(jnp.bfloat16), jnp.uint16).astype(jnp.uint32)
    hi = lax.bitcast_convert_type(
        h2[:, CP2:].astype(jnp.bfloat16), jnp.uint16).astype(jnp.uint32)
    packed = lax.bitcast_convert_type(lo | (hi << 16), jnp.float32)
    out_ref[...] = packed.T


def _out_body(pool_ref, feat_ref, fc3_ref, n2g_ref, n2b_ref, n3g_ref, n3b_ref,
              out_ref):
    u = lax.bitcast_convert_type(pool_ref[...].T, jnp.uint32)
    lo = lax.bitcast_convert_type(
        (u & jnp.uint32(0xFFFF)).astype(jnp.uint16), jnp.bfloat16)
    hi = lax.bitcast_convert_type(
        (u >> 16).astype(jnp.uint16), jnp.bfloat16)
    x = jnp.concatenate(
        [lo.astype(jnp.float32), hi.astype(jnp.float32)], axis=1)
    urow = jnp.full((1, N), 1.0 / N, jnp.float32)
    xn = _bn_relu(x, urow, n2g_ref[...], n2b_ref[...])
    y = jnp.dot(xn, fc3_ref[...], preferred_element_type=jnp.float32)
    m3 = jnp.dot(urow, y, preferred_element_type=jnp.float32,
                 precision=lax.Precision.HIGHEST)
    q3 = jnp.dot(urow, y * y, preferred_element_type=jnp.float32,
                 precision=lax.Precision.HIGHEST)
    scale3 = lax.rsqrt(q3 - m3 * m3 + EPS) * n3g_ref[...]
    shift3 = n3b_ref[...] - m3 * scale3
    out_ref[...] = jnp.maximum(feat_ref[...] + y * scale3 + shift3, 0.0)


def kernel(coord, feat, offset, reference_index, fc1_w, fc3_w, lin1_w, lin1_b,
           lin2_w, lin2_b, n1_g, n1_b, n2_g, n2_b, n3_g, n3_b, m1_g, m1_b,
           m2_g, m2_b):
    bincount, gather_max = _build_sc_kernels()

    idx2d = reference_index.reshape(NW, IDX_PER_W)
    pc = bincount(idx2d)

    r1 = lambda a: a.reshape(1, C)
    h2t = pl.pallas_call(
        _mlp_body,
        out_shape=jax.ShapeDtypeStruct((CP2, N), jnp.float32),
    )(pc, feat, fc1_w.T, lin1_w.T, r1(lin1_b), lin2_w.T, r1(lin2_b),
      r1(n1_g), r1(n1_b), r1(m1_g), r1(m1_b), r1(m2_g), r1(m2_b))

    ri3 = reference_index.reshape(NCHK, CHP, K)
    idxp = (ri3[:, :, 0::2] | (ri3[:, :, 1::2] << 16)).transpose(0, 2, 1)
    pooled = gather_max(h2t, idxp)

    out = pl.pallas_call(
        _out_body,
        out_shape=jax.ShapeDtypeStruct((N, C), jnp.float32),
    )(pooled, feat, fc3_w.T, r1(n2_g), r1(n2_b), r1(n3_g), r1(n3_b))

    return (coord, out, offset)


# speedup vs baseline: 1.1529x; 1.1529x over previous
import functools

import jax
import jax.numpy as jnp
from jax import lax
from jax.experimental import pallas as pl
from jax.experimental.pallas import tpu as pltpu
from jax.experimental.pallas import tpu_sc as plsc

N, K, C = 10000, 32, 128
NC, NS = 2, 16
NW = NC * NS
CP2 = C // 2
CW = CP2 // NW
K2 = K // 2
CHP = 400
NCHK = N // CHP
PB = CHP // 16
IDX_PER_W = (N * K) // NW
EPS = 1e-5


def _wid():
    return lax.axis_index("s") * NC + lax.axis_index("c")


@functools.lru_cache(maxsize=None)
def _build_sc_kernels():
    mesh = plsc.VectorSubcoreMesh(core_axis_name="c", subcore_axis_name="s")
    cparams = pltpu.CompilerParams(needs_layout_passes=False)

    @functools.partial(
        pl.kernel,
        out_type=jax.ShapeDtypeStruct((NW, N), jnp.float32),
        mesh=mesh,
        compiler_params=cparams,
        scratch_types=[
            pltpu.VMEM((IDX_PER_W,), jnp.int32),
            pltpu.VMEM((N,), jnp.float32),
        ],
    )
    def bincount(idx_hbm, pc_hbm, idx_v, cnt_v):
        w = _wid()
        pltpu.sync_copy(idx_hbm.at[w], idx_v)
        zeros = jnp.zeros((16,), jnp.float32)

        def zbody(i, c):
            cnt_v[pl.ds(i * 16, 16)] = zeros
            return c

        lax.fori_loop(0, N // 16, zbody, 0)
        ones = jnp.ones((16,), jnp.float32)

        def abody(i, c):
            v = idx_v[pl.ds(i * 16, 16)]
            plsc.addupdate_scatter(cnt_v, [v], ones)
            return c

        lax.fori_loop(0, IDX_PER_W // 16, abody, 0)
        pltpu.sync_copy(cnt_v, pc_hbm.at[w])

    @functools.partial(
        pl.kernel,
        out_type=jax.ShapeDtypeStruct((CP2, N), jnp.float32),
        mesh=mesh,
        compiler_params=cparams,
        scratch_types=[
            pltpu.VMEM((CW, N), jnp.float32),
            pltpu.VMEM((K2, CHP), jnp.int32),
            pltpu.VMEM((K2, CHP), jnp.int32),
            pltpu.VMEM((CW, N), jnp.float32),
            pltpu.SemaphoreType.DMA,
            pltpu.SemaphoreType.DMA,
        ],
    )
    def gather_max(tblT_hbm, idxc_hbm, outT_hbm, tbl_v, idx0, idx1, out_v,
                   sem0, sem1):
        w = _wid()
        pltpu.sync_copy(tblT_hbm.at[pl.ds(w * CW, CW)], tbl_v)
        pltpu.async_copy(idxc_hbm.at[0], idx0, sem0)
        cvecs = [jnp.full((16,), c, jnp.int32) for c in range(CW)]

        def compute(idx_v, ch):
            n0 = ch * CHP

            mask16 = jnp.full((16,), 0xFFFF, jnp.int32)

            def pbbody(pb, c):
                col = pb * 16
                accs = None
                for k2 in range(K2):
                    vk = idx_v[k2, pl.ds(col, 16)]
                    vlo = jnp.bitwise_and(vk, mask16)
                    vhi = lax.shift_right_logical(vk, 16)
                    for vi in (vlo, vhi):
                        if accs is None:
                            accs = [
                                plsc.bitcast(
                                    plsc.load_gather(tbl_v, [cvecs[c4], vi]),
                                    jnp.bfloat16)
                                for c4 in range(CW)
                            ]
                        else:
                            for c4 in range(CW):
                                g = plsc.bitcast(
                                    plsc.load_gather(tbl_v, [cvecs[c4], vi]),
                                    jnp.bfloat16)
                                accs[c4] = jnp.maximum(accs[c4], g)
                for c4 in range(CW):
                    out_v[c4, pl.ds(n0 + col, 16)] = plsc.bitcast(
                        accs[c4], jnp.float32)
                return c

            lax.fori_loop(0, PB, pbbody, 0)

        def jbody(j, c):
            ch0 = 2 * j
            ch1 = 2 * j + 1
            pltpu.make_async_copy(idxc_hbm.at[ch0], idx0, sem0).wait()
            pltpu.async_copy(idxc_hbm.at[ch1], idx1, sem1)
            compute(idx0, ch0)
            pltpu.async_copy(idxc_hbm.at[ch0 + 2], idx0, sem0)
            pltpu.make_async_copy(idxc_hbm.at[ch1], idx1, sem1).wait()
            compute(idx1, ch1)
            return c

        lax.fori_loop(0, NCHK // 2, jbody, 0)
        pltpu.make_async_copy(idxc_hbm.at[NCHK - 1], idx0, sem0).wait()
        compute(idx0, NCHK - 1)
        pltpu.sync_copy(out_v, outT_hbm.at[pl.ds(w * CW, CW)])

    return bincount, gather_max


def _apply_bn_relu(y, m, q, g, b):
    scale = lax.rsqrt(q - m * m + EPS) * g
    shift = b - m * scale
    return jnp.maximum(y * scale + shift, 0.0)


def _bn_relu_uniform(y, g, b):
    m = jnp.mean(y, axis=0, keepdims=True)
    q = jnp.mean(y * y, axis=0, keepdims=True)
    return _apply_bn_relu(y, m, q, g, b)


def _bn_relu_weighted(y, wrow, g, b):
    m = jnp.dot(wrow, y, preferred_element_type=jnp.float32)
    q = jnp.dot(wrow, y * y, preferred_element_type=jnp.float32)
    return _apply_bn_relu(y, m, q, g, b)


def _mlp_body(pc_ref, feat_ref, fc1_ref, l1w_ref, l1b_ref, l2w_ref, l2b_ref,
              n1g_ref, n1b_ref, m1g_ref, m1b_ref, m2g_ref, m2b_ref, out_ref):
    f = feat_ref[...]
    y0 = jnp.dot(f, fc1_ref[...], preferred_element_type=jnp.float32)
    x1 = _bn_relu_uniform(y0, n1g_ref[...], n1b_ref[...])

    wrow = jnp.sum(pc_ref[...], axis=0, keepdims=True) * (1.0 / (N * K))

    y1 = jnp.dot(x1, l1w_ref[...], preferred_element_type=jnp.float32) + l1b_ref[...]
    h1 = _bn_relu_weighted(y1, wrow, m1g_ref[...], m1b_ref[...])

    y2 = jnp.dot(h1, l2w_ref[...], preferred_element_type=jnp.float32) + l2b_ref[...]
    h2 = _bn_relu_weighted(y2, wrow, m2g_ref[...], m2b_ref[...])
    lo = lax.bitcast_convert_type(
        h2[:, :CP2].astype(jnp.bfloat16), jnp.uint16).astype(jnp.uint32)
    hi = lax.bitcast_convert_type(
        h2[:, CP2:].astype(jnp.bfloat16), jnp.uint16).astype(jnp.uint32)
    packed = lax.bitcast_convert_type(lo | (hi << 16), jnp.float32)
    out_ref[...] = packed.T


def _out_body(pool_ref, feat_ref, fc3_ref, n2g_ref, n2b_ref, n3g_ref, n3b_ref,
              out_ref):
    u = lax.bitcast_convert_type(pool_ref[...].T, jnp.uint32)
    lo = lax.bitcast_convert_type(
        (u & jnp.uint32(0xFFFF)).astype(jnp.uint16), jnp.bfloat16)
    hi = lax.bitcast_convert_type(
        (u >> 16).astype(jnp.uint16), jnp.bfloat16)
    x = jnp.concatenate(
        [lo.astype(jnp.float32), hi.astype(jnp.float32)], axis=1)
    xn = _bn_relu_uniform(x, n2g_ref[...], n2b_ref[...])
    y = jnp.dot(xn, fc3_ref[...], preferred_element_type=jnp.float32)
    m3 = jnp.mean(y, axis=0, keepdims=True)
    q3 = jnp.mean(y * y, axis=0, keepdims=True)
    scale3 = lax.rsqrt(q3 - m3 * m3 + EPS) * n3g_ref[...]
    shift3 = n3b_ref[...] - m3 * scale3
    out_ref[...] = jnp.maximum(feat_ref[...] + y * scale3 + shift3, 0.0)


def kernel(coord, feat, offset, reference_index, fc1_w, fc3_w, lin1_w, lin1_b,
           lin2_w, lin2_b, n1_g, n1_b, n2_g, n2_b, n3_g, n3_b, m1_g, m1_b,
           m2_g, m2_b):
    bincount, gather_max = _build_sc_kernels()

    idx2d = reference_index.reshape(NW, IDX_PER_W)
    pc = bincount(idx2d)

    r1 = lambda a: a.reshape(1, C)
    h2t = pl.pallas_call(
        _mlp_body,
        out_shape=jax.ShapeDtypeStruct((CP2, N), jnp.float32),
    )(pc, feat, fc1_w.T, lin1_w.T, r1(lin1_b), lin2_w.T, r1(lin2_b),
      r1(n1_g), r1(n1_b), r1(m1_g), r1(m1_b), r1(m2_g), r1(m2_b))

    ri3 = reference_index.reshape(NCHK, CHP, K)
    idxp = (ri3[:, :, 0::2] | (ri3[:, :, 1::2] << 16)).transpose(0, 2, 1)
    pooled = gather_max(h2t, idxp)

    out = pl.pallas_call(
        _out_body,
        out_shape=jax.ShapeDtypeStruct((N, C), jnp.float32),
    )(pooled, feat, fc3_w.T, r1(n2_g), r1(n2_b), r1(n3_g), r1(n3_b))

    return (coord, out, offset)
